# Initial kernel scaffold; baseline (speedup 1.0000x reference)
import jax
import jax.numpy as jnp
from jax.experimental import pallas as pl


def kernel(node_features, edge_index, batch_nodes, weight):
    raise NotImplementedError("write your pallas kernel here")


# speedup vs baseline: 19.3610x; 19.3610x over previous
import functools

import jax
import jax.numpy as jnp
from jax import lax
from jax.experimental import pallas as pl
from jax.experimental.pallas import tpu as pltpu
from jax.experimental.pallas import tpu_sc as plsc

N = 10000
D = 128
E = 320000
B = 4096
NC, NS = 2, 16
NW = NC * NS
CW = 128
KCH = 80
EPT = KCH * CW
EPAD = NW * EPT
NTRASH = 240
NROWS = N + NTRASH
ROWS_T = NROWS // NS
XR = 624
BPT = B // NW

_mesh = plsc.VectorSubcoreMesh(
    core_axis_name="c", subcore_axis_name="s", num_cores=NC, num_subcores=NS)


def _deg_body(dst3, z, ones, degp, didx, onesv, shared, sem):
    cid = lax.axis_index("c")
    sid = lax.axis_index("s")
    wid = cid * NS + sid
    pltpu.sync_copy(z.at[pl.ds(sid * ROWS_T, ROWS_T)],
                    shared.at[pl.ds(sid * ROWS_T, ROWS_T)])
    pltpu.sync_copy(dst3.at[wid], didx)
    pltpu.sync_copy(ones, onesv)
    plsc.subcore_barrier()

    def step(k, _):
        pltpu.async_copy(onesv, shared.at[didx.at[k]], sem, add=True).wait()
        return _
    lax.fori_loop(0, KCH, step, None)
    plsc.subcore_barrier()
    pltpu.sync_copy(shared.at[pl.ds(sid * ROWS_T, ROWS_T)],
                    degp.at[cid, pl.ds(sid * ROWS_T, ROWS_T)])


_deg_kernel = functools.partial(
    pl.kernel,
    out_type=jax.ShapeDtypeStruct((NC, NROWS, D), jnp.float32),
    mesh=_mesh,
    scratch_types=[
        pltpu.VMEM((KCH, CW), jnp.int32),
        pltpu.VMEM((CW, D), jnp.float32),
        pltpu.VMEM_SHARED((NROWS, D), jnp.float32),
        pltpu.SemaphoreType.DMA,
    ],
)(_deg_body)


def _scale_body(degp_ref, x_ref, w_ref, y_ref, c1_ref):
    deg = degp_ref[0, :N, 0:1] + degp_ref[1, :N, 0:1]
    c = lax.rsqrt(jnp.maximum(deg, 1.0))
    xw = jnp.dot(x_ref[...], w_ref[...], preferred_element_type=jnp.float32,
                 precision=lax.Precision.HIGHEST)
    y_ref[...] = c * xw
    c1_ref[...] = c[:, 0]


def _agg_body(y, x, src3, dst3, z, p0, p1, sidx, didx, rbuf, shared, sem):
    cid = lax.axis_index("c")
    sid = lax.axis_index("s")
    wid = cid * NS + sid

    @pl.when(cid == 0)
    def _():
        pltpu.sync_copy(x.at[pl.ds(sid * XR, XR)],
                        shared.at[pl.ds(sid * XR, XR)])

    @pl.when((cid == 0) & (sid == 0))
    def _():
        pltpu.sync_copy(x.at[pl.ds(NS * XR, N - NS * XR)],
                        shared.at[pl.ds(NS * XR, N - NS * XR)])

    @pl.when(cid == 1)
    def _():
        pltpu.sync_copy(z.at[pl.ds(sid * ROWS_T, ROWS_T)],
                        shared.at[pl.ds(sid * ROWS_T, ROWS_T)])

    pltpu.sync_copy(src3.at[wid], sidx)
    pltpu.sync_copy(dst3.at[wid], didx)
    plsc.subcore_barrier()

    def step(k, _):
        pltpu.async_copy(y.at[sidx.at[k]], rbuf, sem).wait()
        pltpu.async_copy(rbuf, shared.at[didx.at[k]], sem, add=True).wait()
        return _
    lax.fori_loop(0, KCH, step, None)
    plsc.subcore_barrier()

    @pl.when(cid == 0)
    def _():
        pltpu.sync_copy(shared.at[pl.ds(sid * ROWS_T, ROWS_T)],
                        p0.at[pl.ds(sid * ROWS_T, ROWS_T)])

    @pl.when(cid == 1)
    def _():
        pltpu.sync_copy(shared.at[pl.ds(sid * ROWS_T, ROWS_T)],
                        p1.at[pl.ds(sid * ROWS_T, ROWS_T)])


_agg_kernel = functools.partial(
    pl.kernel,
    out_type=(jax.ShapeDtypeStruct((NROWS, D), jnp.float32),
              jax.ShapeDtypeStruct((NROWS, D), jnp.float32)),
    mesh=_mesh,
    scratch_types=[
        pltpu.VMEM((KCH, CW), jnp.int32),
        pltpu.VMEM((KCH, CW), jnp.int32),
        pltpu.VMEM((CW, D), jnp.float32),
        pltpu.VMEM_SHARED((NROWS, D), jnp.float32),
        pltpu.SemaphoreType.DMA,
    ],
)(_agg_body)


def _gather_body(h, bn, out, bidx, brows, sem):
    cid = lax.axis_index("c")
    sid = lax.axis_index("s")
    wid = cid * NS + sid
    base = wid * BPT
    pltpu.sync_copy(bn.at[pl.ds(base, BPT)], bidx)
    pltpu.async_copy(h.at[bidx], brows, sem).wait()
    pltpu.sync_copy(brows, out.at[pl.ds(base, BPT)])


_gather_kernel = functools.partial(
    pl.kernel,
    out_type=jax.ShapeDtypeStruct((B, D), jnp.float32),
    mesh=_mesh,
    scratch_types=[
        pltpu.VMEM((BPT,), jnp.int32),
        pltpu.VMEM((BPT, D), jnp.float32),
        pltpu.SemaphoreType.DMA,
    ],
)(_gather_body)


def _final_body(p0_ref, p1_ref, c1_ref, h_ref):
    h_ref[...] = jnp.maximum(
        c1_ref[...][:, None] * (p0_ref[:N] + p1_ref[:N]), 0.0)


def kernel(node_features, edge_index, batch_nodes, weight):
    src = edge_index[0]
    dst = edge_index[1]
    pad = EPAD - E
    ar = jnp.arange(pad, dtype=jnp.int32)
    src_p = jnp.concatenate([src, (ar * 13) % N]).reshape(NW, KCH, CW)
    dst_p = jnp.concatenate([dst, N + (ar % NTRASH)]).reshape(NW, KCH, CW)
    ones = jnp.ones((CW, D), jnp.float32)
    z = jnp.zeros((NROWS, D), jnp.float32)

    degp = _deg_kernel(dst_p, z, ones)

    y, c1 = pl.pallas_call(
        _scale_body,
        out_shape=(jax.ShapeDtypeStruct((N, D), jnp.float32),
                   jax.ShapeDtypeStruct((N,), jnp.float32)),
    )(degp, node_features, weight)

    p0, p1 = _agg_kernel(y, node_features, src_p, dst_p, z)

    h = pl.pallas_call(
        _final_body,
        out_shape=jax.ShapeDtypeStruct((N, D), jnp.float32),
    )(p0, p1, c1)

    return _gather_kernel(h, batch_nodes)


# speedup vs baseline: 22.5689x; 1.1657x over previous
import functools

import jax
import jax.numpy as jnp
from jax import lax
from jax.experimental import pallas as pl
from jax.experimental.pallas import tpu as pltpu
from jax.experimental.pallas import tpu_sc as plsc

N = 10000
D = 128
E = 320000
B = 4096
NC, NS = 2, 16
NW = NC * NS
CW = 128
KCH = 80
HK = 40
EPT = KCH * CW
EPAD = NW * EPT
NTRASH = 240
NROWS = N + NTRASH
ROWS_T = NROWS // NS
XR = 624
BPT = B // NW

_mesh = plsc.VectorSubcoreMesh(
    core_axis_name="c", subcore_axis_name="s", num_cores=NC, num_subcores=NS)


def _deg_body(dst3, z, ones, degp, didx, onesv, shared, sem):
    cid = lax.axis_index("c")
    sid = lax.axis_index("s")
    wid = cid * NS + sid
    pltpu.sync_copy(z.at[pl.ds(sid * ROWS_T, ROWS_T)],
                    shared.at[pl.ds(sid * ROWS_T, ROWS_T)])
    pltpu.sync_copy(dst3.at[wid], didx)
    pltpu.sync_copy(ones, onesv)
    plsc.subcore_barrier()

    F = 8
    for j in range(F):
        pltpu.async_copy(onesv, shared.at[didx.at[j]], sem, add=True)

    def step(k, _):
        pltpu.make_async_copy(onesv, shared.at[didx.at[0]], sem).wait()
        pltpu.async_copy(onesv, shared.at[didx.at[k + F]], sem, add=True)
        return _
    lax.fori_loop(0, KCH - F, step, None)
    for j in range(F):
        pltpu.make_async_copy(onesv, shared.at[didx.at[j]], sem).wait()
    plsc.subcore_barrier()
    pltpu.sync_copy(shared.at[pl.ds(sid * ROWS_T, ROWS_T)],
                    degp.at[cid, pl.ds(sid * ROWS_T, ROWS_T)])


_deg_kernel = functools.partial(
    pl.kernel,
    out_type=jax.ShapeDtypeStruct((NC, NROWS, D), jnp.float32),
    mesh=_mesh,
    scratch_types=[
        pltpu.VMEM((KCH, CW), jnp.int32),
        pltpu.VMEM((CW, D), jnp.float32),
        pltpu.VMEM_SHARED((NROWS, D), jnp.float32),
        pltpu.SemaphoreType.DMA,
    ],
)(_deg_body)


def _scale_body(degp_ref, x_ref, w_ref, y_ref, c1_ref):
    deg = degp_ref[0, :N, 0:1] + degp_ref[1, :N, 0:1]
    c = lax.rsqrt(jnp.maximum(deg, 1.0))
    xw = jnp.dot(x_ref[...], w_ref[...], preferred_element_type=jnp.float32,
                 precision=lax.Precision.HIGHEST)
    y_ref[...] = c * xw
    c1_ref[...] = c[:, 0]


def _agg_body(y, x, src3, dst3, z, p0, p1,
              sidx, didx, rbuf0, rbuf1, shared, gsem0, gsem1):
    cid = lax.axis_index("c")
    sid = lax.axis_index("s")
    wid = cid * NS + sid

    @pl.when(cid == 0)
    def _():
        pltpu.sync_copy(x.at[pl.ds(sid * XR, XR)],
                        shared.at[pl.ds(sid * XR, XR)])

    @pl.when((cid == 0) & (sid == 0))
    def _():
        pltpu.sync_copy(x.at[pl.ds(NS * XR, N - NS * XR)],
                        shared.at[pl.ds(NS * XR, N - NS * XR)])

    @pl.when(cid == 1)
    def _():
        pltpu.sync_copy(z.at[pl.ds(sid * ROWS_T, ROWS_T)],
                        shared.at[pl.ds(sid * ROWS_T, ROWS_T)])

    plsc.subcore_barrier()

    for h in range(KCH // HK):
        pltpu.sync_copy(src3.at[wid, pl.ds(h * HK, HK)], sidx)
        pltpu.sync_copy(dst3.at[wid, pl.ds(h * HK, HK)], didx)
        pltpu.async_copy(y.at[sidx.at[0]], rbuf0, gsem0)

        def step(g, _):
            k0 = 2 * g
            k1 = 2 * g + 1
            knext = jnp.minimum(k1 + 1, HK - 1)
            pltpu.make_async_copy(y.at[sidx.at[k0]], rbuf0, gsem0).wait()
            pltpu.async_copy(y.at[sidx.at[k1]], rbuf1, gsem1)
            pltpu.sync_copy(rbuf0, shared.at[didx.at[k0]], add=True)
            pltpu.make_async_copy(y.at[sidx.at[k1]], rbuf1, gsem1).wait()
            pltpu.async_copy(y.at[sidx.at[knext]], rbuf0, gsem0)
            pltpu.sync_copy(rbuf1, shared.at[didx.at[k1]], add=True)
            return _
        lax.fori_loop(0, HK // 2, step, None)
        pltpu.make_async_copy(y.at[sidx.at[HK - 1]], rbuf0, gsem0).wait()
    plsc.subcore_barrier()

    @pl.when(cid == 0)
    def _():
        pltpu.sync_copy(shared.at[pl.ds(sid * ROWS_T, ROWS_T)],
                        p0.at[pl.ds(sid * ROWS_T, ROWS_T)])

    @pl.when(cid == 1)
    def _():
        pltpu.sync_copy(shared.at[pl.ds(sid * ROWS_T, ROWS_T)],
                        p1.at[pl.ds(sid * ROWS_T, ROWS_T)])


_agg_kernel = functools.partial(
    pl.kernel,
    out_type=(jax.ShapeDtypeStruct((NROWS, D), jnp.float32),
              jax.ShapeDtypeStruct((NROWS, D), jnp.float32)),
    mesh=_mesh,
    scratch_types=[
        pltpu.VMEM((HK, CW), jnp.int32),
        pltpu.VMEM((HK, CW), jnp.int32),
        pltpu.VMEM((CW, D), jnp.float32),
        pltpu.VMEM((CW, D), jnp.float32),
        pltpu.VMEM_SHARED((NROWS, D), jnp.float32),
        pltpu.SemaphoreType.DMA,
        pltpu.SemaphoreType.DMA,
    ],
)(_agg_body)


def _gather_body(h, bn, out, bidx, brows, sem):
    cid = lax.axis_index("c")
    sid = lax.axis_index("s")
    wid = cid * NS + sid
    base = wid * BPT
    pltpu.sync_copy(bn.at[pl.ds(base, BPT)], bidx)
    pltpu.async_copy(h.at[bidx], brows, sem).wait()
    pltpu.sync_copy(brows, out.at[pl.ds(base, BPT)])


_gather_kernel = functools.partial(
    pl.kernel,
    out_type=jax.ShapeDtypeStruct((B, D), jnp.float32),
    mesh=_mesh,
    scratch_types=[
        pltpu.VMEM((BPT,), jnp.int32),
        pltpu.VMEM((BPT, D), jnp.float32),
        pltpu.SemaphoreType.DMA,
    ],
)(_gather_body)


def _final_body(p0_ref, p1_ref, c1_ref, h_ref):
    h_ref[...] = jnp.maximum(
        c1_ref[...][:, None] * (p0_ref[:N] + p1_ref[:N]), 0.0)


def kernel(node_features, edge_index, batch_nodes, weight):
    src = edge_index[0]
    dst = edge_index[1]
    pad = EPAD - E
    ar = jnp.arange(pad, dtype=jnp.int32)
    src_p = jnp.concatenate([src, (ar * 13) % N]).reshape(NW, KCH, CW)
    dst_p = jnp.concatenate([dst, N + (ar % NTRASH)]).reshape(NW, KCH, CW)
    ones = jnp.ones((CW, D), jnp.float32)
    z = jnp.zeros((NROWS, D), jnp.float32)

    degp = _deg_kernel(dst_p, z, ones)

    y, c1 = pl.pallas_call(
        _scale_body,
        out_shape=(jax.ShapeDtypeStruct((N, D), jnp.float32),
                   jax.ShapeDtypeStruct((N,), jnp.float32)),
    )(degp, node_features, weight)

    p0, p1 = _agg_kernel(y, node_features, src_p, dst_p, z)

    h = pl.pallas_call(
        _final_body,
        out_shape=jax.ShapeDtypeStruct((N, D), jnp.float32),
    )(p0, p1, c1)

    return _gather_kernel(h, batch_nodes)


# speedup vs baseline: 27.6046x; 1.2231x over previous
import functools

import jax
import jax.numpy as jnp
from jax import lax
from jax.experimental import pallas as pl
from jax.experimental.pallas import tpu as pltpu
from jax.experimental.pallas import tpu_sc as plsc

N = 10000
D = 128
E = 320000
B = 4096
NC, NS = 2, 16
NW = NC * NS
CW = 128
KCH = 80
HK = 40
DW = 16
EPT = KCH * CW
EPAD = NW * EPT
NTRASH = 240
NROWS = N + NTRASH
ROWS_T = NROWS // NS
XR = 624
BPT = B // NW

_mesh = plsc.VectorSubcoreMesh(
    core_axis_name="c", subcore_axis_name="s", num_cores=NC, num_subcores=NS)


def _deg_body(dst3, z, ones, degp, didx, onesv, shared, sem):
    cid = lax.axis_index("c")
    sid = lax.axis_index("s")
    wid = cid * NS + sid
    pltpu.sync_copy(z.at[pl.ds(sid * ROWS_T, ROWS_T)],
                    shared.at[pl.ds(sid * ROWS_T, ROWS_T)])
    pltpu.sync_copy(dst3.at[wid], didx)
    pltpu.sync_copy(ones, onesv)
    plsc.subcore_barrier()

    F = 8
    for j in range(F):
        pltpu.async_copy(onesv, shared.at[didx.at[j]], sem, add=True)

    def step(k, _):
        pltpu.make_async_copy(onesv, shared.at[didx.at[0]], sem).wait()
        pltpu.async_copy(onesv, shared.at[didx.at[k + F]], sem, add=True)
        return _
    lax.fori_loop(0, KCH - F, step, None)
    for j in range(F):
        pltpu.make_async_copy(onesv, shared.at[didx.at[j]], sem).wait()
    plsc.subcore_barrier()
    pltpu.sync_copy(shared.at[pl.ds(sid * ROWS_T, ROWS_T)],
                    degp.at[cid, pl.ds(sid * ROWS_T, ROWS_T)])


_deg_kernel = functools.partial(
    pl.kernel,
    out_type=jax.ShapeDtypeStruct((NC, NROWS, DW), jnp.float32),
    mesh=_mesh,
    scratch_types=[
        pltpu.VMEM((KCH, CW), jnp.int32),
        pltpu.VMEM((CW, DW), jnp.float32),
        pltpu.VMEM_SHARED((NROWS, DW), jnp.float32),
        pltpu.SemaphoreType.DMA,
    ],
    compiler_params=pltpu.CompilerParams(use_tc_tiling_on_sc=False),
)(_deg_body)


def _scale_body(degp_ref, x_ref, w_ref, y_ref, c1_ref):
    deg = degp_ref[0, :N, 0:1] + degp_ref[1, :N, 0:1]
    c = lax.rsqrt(jnp.maximum(deg, 1.0))
    xw = jnp.dot(x_ref[...], w_ref[...], preferred_element_type=jnp.float32,
                 precision=lax.Precision.HIGHEST)
    y_ref[...] = c * xw
    c1_ref[...] = c[:, 0]


def _agg_body(y, x, src3, dst3, z, p0, p1,
              sidx, didx, rbuf0, rbuf1, shared, gsem0, gsem1):
    cid = lax.axis_index("c")
    sid = lax.axis_index("s")
    wid = cid * NS + sid

    @pl.when(cid == 0)
    def _():
        pltpu.sync_copy(x.at[pl.ds(sid * XR, XR)],
                        shared.at[pl.ds(sid * XR, XR)])

    @pl.when((cid == 0) & (sid == 0))
    def _():
        pltpu.sync_copy(x.at[pl.ds(NS * XR, N - NS * XR)],
                        shared.at[pl.ds(NS * XR, N - NS * XR)])

    @pl.when(cid == 1)
    def _():
        pltpu.sync_copy(z.at[pl.ds(sid * ROWS_T, ROWS_T)],
                        shared.at[pl.ds(sid * ROWS_T, ROWS_T)])

    plsc.subcore_barrier()

    for h in range(KCH // HK):
        pltpu.sync_copy(src3.at[wid, pl.ds(h * HK, HK)], sidx)
        pltpu.sync_copy(dst3.at[wid, pl.ds(h * HK, HK)], didx)
        pltpu.async_copy(y.at[sidx.at[0]], rbuf0, gsem0)

        def step(g, _):
            k0 = 2 * g
            k1 = 2 * g + 1
            knext = jnp.minimum(k1 + 1, HK - 1)
            pltpu.make_async_copy(y.at[sidx.at[k0]], rbuf0, gsem0).wait()
            pltpu.async_copy(y.at[sidx.at[k1]], rbuf1, gsem1)
            pltpu.sync_copy(rbuf0, shared.at[didx.at[k0]], add=True)
            pltpu.make_async_copy(y.at[sidx.at[k1]], rbuf1, gsem1).wait()
            pltpu.async_copy(y.at[sidx.at[knext]], rbuf0, gsem0)
            pltpu.sync_copy(rbuf1, shared.at[didx.at[k1]], add=True)
            return _
        lax.fori_loop(0, HK // 2, step, None)
        pltpu.make_async_copy(y.at[sidx.at[HK - 1]], rbuf0, gsem0).wait()
    plsc.subcore_barrier()

    @pl.when(cid == 0)
    def _():
        pltpu.sync_copy(shared.at[pl.ds(sid * ROWS_T, ROWS_T)],
                        p0.at[pl.ds(sid * ROWS_T, ROWS_T)])

    @pl.when(cid == 1)
    def _():
        pltpu.sync_copy(shared.at[pl.ds(sid * ROWS_T, ROWS_T)],
                        p1.at[pl.ds(sid * ROWS_T, ROWS_T)])


_agg_kernel = functools.partial(
    pl.kernel,
    out_type=(jax.ShapeDtypeStruct((NROWS, D), jnp.float32),
              jax.ShapeDtypeStruct((NROWS, D), jnp.float32)),
    mesh=_mesh,
    scratch_types=[
        pltpu.VMEM((HK, CW), jnp.int32),
        pltpu.VMEM((HK, CW), jnp.int32),
        pltpu.VMEM((CW, D), jnp.float32),
        pltpu.VMEM((CW, D), jnp.float32),
        pltpu.VMEM_SHARED((NROWS, D), jnp.float32),
        pltpu.SemaphoreType.DMA,
        pltpu.SemaphoreType.DMA,
    ],
)(_agg_body)


def _gather_body(h, bn, out, bidx, brows, sem):
    cid = lax.axis_index("c")
    sid = lax.axis_index("s")
    wid = cid * NS + sid
    base = wid * BPT
    pltpu.sync_copy(bn.at[pl.ds(base, BPT)], bidx)
    pltpu.async_copy(h.at[bidx], brows, sem).wait()
    pltpu.sync_copy(brows, out.at[pl.ds(base, BPT)])


_gather_kernel = functools.partial(
    pl.kernel,
    out_type=jax.ShapeDtypeStruct((B, D), jnp.float32),
    mesh=_mesh,
    scratch_types=[
        pltpu.VMEM((BPT,), jnp.int32),
        pltpu.VMEM((BPT, D), jnp.float32),
        pltpu.SemaphoreType.DMA,
    ],
)(_gather_body)


def _final_body(p0_ref, p1_ref, c1_ref, h_ref):
    h_ref[...] = jnp.maximum(
        c1_ref[...][:, None] * (p0_ref[:N] + p1_ref[:N]), 0.0)


def kernel(node_features, edge_index, batch_nodes, weight):
    src = edge_index[0]
    dst = edge_index[1]
    pad = EPAD - E
    ar = jnp.arange(pad, dtype=jnp.int32)
    src_p = jnp.concatenate([src, (ar * 13) % N]).reshape(NW, KCH, CW)
    dst_p = jnp.concatenate([dst, N + (ar % NTRASH)]).reshape(NW, KCH, CW)
    ones = jnp.ones((CW, DW), jnp.float32)
    z16 = jnp.zeros((NROWS, DW), jnp.float32)
    z = jnp.zeros((NROWS, D), jnp.float32)

    degp = _deg_kernel(dst_p, z16, ones)

    y, c1 = pl.pallas_call(
        _scale_body,
        out_shape=(jax.ShapeDtypeStruct((N, D), jnp.float32),
                   jax.ShapeDtypeStruct((N,), jnp.float32)),
    )(degp, node_features, weight)

    p0, p1 = _agg_kernel(y, node_features, src_p, dst_p, z)

    h = pl.pallas_call(
        _final_body,
        out_shape=jax.ShapeDtypeStruct((N, D), jnp.float32),
    )(p0, p1, c1)

    return _gather_kernel(h, batch_nodes)


# speedup vs baseline: 29.1625x; 1.0564x over previous
import functools

import jax
import jax.numpy as jnp
from jax import lax
from jax.experimental import pallas as pl
from jax.experimental.pallas import tpu as pltpu
from jax.experimental.pallas import tpu_sc as plsc

N = 10000
D = 128
E = 320000
B = 4096
NC, NS = 2, 16
NW = NC * NS
CW = 128
KCH = 80
HK = 40
DW = 16
EPT = KCH * CW
EPAD = NW * EPT
NTRASH = 240
NROWS = N + NTRASH
ROWS_T = NROWS // NS
XR = 624
BPT = B // NW

_mesh = plsc.VectorSubcoreMesh(
    core_axis_name="c", subcore_axis_name="s", num_cores=NC, num_subcores=NS)


def _deg_body(dst3, z, ones, degp, didx, onesv, shared, sem):
    cid = lax.axis_index("c")
    sid = lax.axis_index("s")
    wid = cid * NS + sid
    pltpu.sync_copy(z.at[pl.ds(sid * ROWS_T, ROWS_T)],
                    shared.at[pl.ds(sid * ROWS_T, ROWS_T)])
    pltpu.sync_copy(dst3.at[wid], didx)
    pltpu.sync_copy(ones, onesv)
    plsc.subcore_barrier()

    F = 8
    for j in range(F):
        pltpu.async_copy(onesv, shared.at[didx.at[j]], sem, add=True)

    def step(k, _):
        pltpu.make_async_copy(onesv, shared.at[didx.at[0]], sem).wait()
        pltpu.async_copy(onesv, shared.at[didx.at[k + F]], sem, add=True)
        return _
    lax.fori_loop(0, KCH - F, step, None)
    for j in range(F):
        pltpu.make_async_copy(onesv, shared.at[didx.at[j]], sem).wait()
    plsc.subcore_barrier()
    pltpu.sync_copy(shared.at[pl.ds(sid * ROWS_T, ROWS_T)],
                    degp.at[cid, pl.ds(sid * ROWS_T, ROWS_T)])


_deg_kernel = functools.partial(
    pl.kernel,
    out_type=jax.ShapeDtypeStruct((NC, NROWS, DW), jnp.float32),
    mesh=_mesh,
    scratch_types=[
        pltpu.VMEM((KCH, CW), jnp.int32),
        pltpu.VMEM((CW, DW), jnp.float32),
        pltpu.VMEM_SHARED((NROWS, DW), jnp.float32),
        pltpu.SemaphoreType.DMA,
    ],
    compiler_params=pltpu.CompilerParams(use_tc_tiling_on_sc=False),
)(_deg_body)


def _scale_body(degp_ref, x_ref, w_ref, y_ref, c1_ref):
    deg = degp_ref[0, :N, 0:1] + degp_ref[1, :N, 0:1]
    c = lax.rsqrt(jnp.maximum(deg, 1.0))
    xw = jnp.dot(x_ref[...], w_ref[...], preferred_element_type=jnp.float32,
                 precision=lax.Precision.HIGHEST)
    y_ref[...] = (c * xw).astype(jnp.bfloat16)
    c1_ref[...] = c[:, 0]


def _agg_body(y, x, src3, dst3, z, p0, p1,
              sidx, didx, rbuf0, rbuf1, shared, gsem0, gsem1):
    cid = lax.axis_index("c")
    sid = lax.axis_index("s")
    wid = cid * NS + sid

    @pl.when(cid == 0)
    def _():
        pltpu.sync_copy(x.at[pl.ds(sid * XR, XR)],
                        shared.at[pl.ds(sid * XR, XR)])

    @pl.when((cid == 0) & (sid == 0))
    def _():
        pltpu.sync_copy(x.at[pl.ds(NS * XR, N - NS * XR)],
                        shared.at[pl.ds(NS * XR, N - NS * XR)])

    @pl.when(cid == 1)
    def _():
        pltpu.sync_copy(z.at[pl.ds(sid * ROWS_T, ROWS_T)],
                        shared.at[pl.ds(sid * ROWS_T, ROWS_T)])

    plsc.subcore_barrier()

    for h in range(KCH // HK):
        pltpu.sync_copy(src3.at[wid, pl.ds(h * HK, HK)], sidx)
        pltpu.sync_copy(dst3.at[wid, pl.ds(h * HK, HK)], didx)
        pltpu.async_copy(y.at[sidx.at[0]], rbuf0, gsem0)

        def step(g, _):
            k0 = 2 * g
            k1 = 2 * g + 1
            knext = jnp.minimum(k1 + 1, HK - 1)
            pltpu.make_async_copy(y.at[sidx.at[k0]], rbuf0, gsem0).wait()
            pltpu.async_copy(y.at[sidx.at[k1]], rbuf1, gsem1)
            pltpu.sync_copy(rbuf0, shared.at[didx.at[k0]], add=True)
            pltpu.make_async_copy(y.at[sidx.at[k1]], rbuf1, gsem1).wait()
            pltpu.async_copy(y.at[sidx.at[knext]], rbuf0, gsem0)
            pltpu.sync_copy(rbuf1, shared.at[didx.at[k1]], add=True)
            return _
        lax.fori_loop(0, HK // 2, step, None)
        pltpu.make_async_copy(y.at[sidx.at[HK - 1]], rbuf0, gsem0).wait()
    plsc.subcore_barrier()

    @pl.when(cid == 0)
    def _():
        pltpu.sync_copy(shared.at[pl.ds(sid * ROWS_T, ROWS_T)],
                        p0.at[pl.ds(sid * ROWS_T, ROWS_T)])

    @pl.when(cid == 1)
    def _():
        pltpu.sync_copy(shared.at[pl.ds(sid * ROWS_T, ROWS_T)],
                        p1.at[pl.ds(sid * ROWS_T, ROWS_T)])


_agg_kernel = functools.partial(
    pl.kernel,
    out_type=(jax.ShapeDtypeStruct((NROWS, D), jnp.bfloat16),
              jax.ShapeDtypeStruct((NROWS, D), jnp.bfloat16)),
    mesh=_mesh,
    scratch_types=[
        pltpu.VMEM((HK, CW), jnp.int32),
        pltpu.VMEM((HK, CW), jnp.int32),
        pltpu.VMEM((CW, D), jnp.bfloat16),
        pltpu.VMEM((CW, D), jnp.bfloat16),
        pltpu.VMEM_SHARED((NROWS, D), jnp.bfloat16),
        pltpu.SemaphoreType.DMA,
        pltpu.SemaphoreType.DMA,
    ],
    compiler_params=pltpu.CompilerParams(use_tc_tiling_on_sc=False),
)(_agg_body)


def _gather_body(h, bn, out, bidx, brows, sem):
    cid = lax.axis_index("c")
    sid = lax.axis_index("s")
    wid = cid * NS + sid
    base = wid * BPT
    pltpu.sync_copy(bn.at[pl.ds(base, BPT)], bidx)
    pltpu.async_copy(h.at[bidx], brows, sem).wait()
    pltpu.sync_copy(brows, out.at[pl.ds(base, BPT)])


_gather_kernel = functools.partial(
    pl.kernel,
    out_type=jax.ShapeDtypeStruct((B, D), jnp.float32),
    mesh=_mesh,
    scratch_types=[
        pltpu.VMEM((BPT,), jnp.int32),
        pltpu.VMEM((BPT, D), jnp.float32),
        pltpu.SemaphoreType.DMA,
    ],
)(_gather_body)


def _final_body(p0_ref, p1_ref, c1_ref, h_ref):
    s = p0_ref[:N].astype(jnp.float32) + p1_ref[:N].astype(jnp.float32)
    h_ref[...] = jnp.maximum(c1_ref[...][:, None] * s, 0.0)


def kernel(node_features, edge_index, batch_nodes, weight):
    src = edge_index[0]
    dst = edge_index[1]
    pad = EPAD - E
    ar = jnp.arange(pad, dtype=jnp.int32)
    src_p = jnp.concatenate([src, (ar * 13) % N]).reshape(NW, KCH, CW)
    dst_p = jnp.concatenate([dst, N + (ar % NTRASH)]).reshape(NW, KCH, CW)
    ones = jnp.ones((CW, DW), jnp.float32)
    z16 = jnp.zeros((NROWS, DW), jnp.float32)
    z = jnp.zeros((NROWS, D), jnp.bfloat16)

    degp = _deg_kernel(dst_p, z16, ones)

    y, c1 = pl.pallas_call(
        _scale_body,
        out_shape=(jax.ShapeDtypeStruct((N, D), jnp.bfloat16),
                   jax.ShapeDtypeStruct((N,), jnp.float32)),
    )(degp, node_features, weight)

    xb = node_features.astype(jnp.bfloat16)
    p0, p1 = _agg_kernel(y, xb, src_p, dst_p, z)

    h = pl.pallas_call(
        _final_body,
        out_shape=jax.ShapeDtypeStruct((N, D), jnp.float32),
    )(p0, p1, c1)

    return _gather_kernel(h, batch_nodes)


# speedup vs baseline: 33.4385x; 1.1466x over previous
import functools

import jax
import jax.numpy as jnp
from jax import lax
from jax.experimental import pallas as pl
from jax.experimental.pallas import tpu as pltpu
from jax.experimental.pallas import tpu_sc as plsc

N = 10000
D = 128
E = 320000
B = 4096
NC, NS = 2, 16
NW = NC * NS
CW = 128
KCH = 80
HK = 40
DW = 16
EPT = KCH * CW
EPAD = NW * EPT
NTRASH = 240
NROWS = N + NTRASH
ROWS_T = NROWS // NS
XR = 624
BPT = B // NW

_mesh = plsc.VectorSubcoreMesh(
    core_axis_name="c", subcore_axis_name="s", num_cores=NC, num_subcores=NS)


def _deg_body(dst3, z, ones, degp, didx, onesv, shared, sem):
    cid = lax.axis_index("c")
    sid = lax.axis_index("s")
    wid = cid * NS + sid
    pltpu.sync_copy(z.at[pl.ds(sid * ROWS_T, ROWS_T)],
                    shared.at[pl.ds(sid * ROWS_T, ROWS_T)])
    pltpu.sync_copy(dst3.at[wid], didx)
    pltpu.sync_copy(ones, onesv)
    plsc.subcore_barrier()

    F = 8
    for j in range(F):
        pltpu.async_copy(onesv, shared.at[didx.at[j]], sem, add=True)

    def step(k, _):
        pltpu.make_async_copy(onesv, shared.at[didx.at[0]], sem).wait()
        pltpu.async_copy(onesv, shared.at[didx.at[k + F]], sem, add=True)
        return _
    lax.fori_loop(0, KCH - F, step, None)
    for j in range(F):
        pltpu.make_async_copy(onesv, shared.at[didx.at[j]], sem).wait()
    plsc.subcore_barrier()
    pltpu.sync_copy(shared.at[pl.ds(sid * ROWS_T, ROWS_T)],
                    degp.at[cid, pl.ds(sid * ROWS_T, ROWS_T)])


_deg_kernel = functools.partial(
    pl.kernel,
    out_type=jax.ShapeDtypeStruct((NC, NROWS, DW), jnp.float32),
    mesh=_mesh,
    scratch_types=[
        pltpu.VMEM((KCH, CW), jnp.int32),
        pltpu.VMEM((CW, DW), jnp.float32),
        pltpu.VMEM_SHARED((NROWS, DW), jnp.float32),
        pltpu.SemaphoreType.DMA,
    ],
    compiler_params=pltpu.CompilerParams(use_tc_tiling_on_sc=False),
)(_deg_body)


def _scale_body(degp_ref, x_ref, w_ref, y_ref, c1_ref):
    deg = degp_ref[0, :N, 0:1] + degp_ref[1, :N, 0:1]
    c = lax.rsqrt(jnp.maximum(deg, 1.0))
    xw = jnp.dot(x_ref[...], w_ref[...], preferred_element_type=jnp.float32,
                 precision=lax.Precision.HIGHEST)
    y_ref[...] = (c * xw).astype(jnp.bfloat16)
    c1_ref[...] = c[:, 0]


def _agg_body(y, x, src3, dst3, z, p0, p1,
              sidx, didx, rb0, rb1, rb2, rb3, shared,
              gs0, gs1, gs2, gs3, ss0, ss1, ss2, ss3):
    rb = (rb0, rb1, rb2, rb3)
    gs = (gs0, gs1, gs2, gs3)
    ss = (ss0, ss1, ss2, ss3)
    cid = lax.axis_index("c")
    sid = lax.axis_index("s")
    wid = cid * NS + sid

    @pl.when(cid == 0)
    def _():
        pltpu.sync_copy(x.at[pl.ds(sid * XR, XR)],
                        shared.at[pl.ds(sid * XR, XR)])

    @pl.when((cid == 0) & (sid == 0))
    def _():
        pltpu.sync_copy(x.at[pl.ds(NS * XR, N - NS * XR)],
                        shared.at[pl.ds(NS * XR, N - NS * XR)])

    @pl.when(cid == 1)
    def _():
        pltpu.sync_copy(z.at[pl.ds(sid * ROWS_T, ROWS_T)],
                        shared.at[pl.ds(sid * ROWS_T, ROWS_T)])

    plsc.subcore_barrier()

    for h in range(KCH // HK):
        pltpu.sync_copy(src3.at[wid, pl.ds(h * HK, HK)], sidx)
        pltpu.sync_copy(dst3.at[wid, pl.ds(h * HK, HK)], didx)
        for j in range(4):
            pltpu.async_copy(y.at[sidx.at[j]], rb[j], gs[j])

        def step(g, _):
            for j in range(4):
                k = 4 * g + j
                pltpu.make_async_copy(y.at[sidx.at[k]], rb[j], gs[j]).wait()
                pltpu.async_copy(rb[j], shared.at[didx.at[k]], ss[j], add=True)
            for j in range(4):
                knext = jnp.minimum(4 * g + 4 + j, HK - 1)
                pltpu.make_async_copy(rb[j], shared.at[didx.at[0]], ss[j]).wait()
                pltpu.async_copy(y.at[sidx.at[knext]], rb[j], gs[j])
            return _
        lax.fori_loop(0, HK // 4, step, None)
        for j in range(4):
            pltpu.make_async_copy(y.at[sidx.at[HK - 1]], rb[j], gs[j]).wait()
    plsc.subcore_barrier()

    @pl.when(cid == 0)
    def _():
        pltpu.sync_copy(shared.at[pl.ds(sid * ROWS_T, ROWS_T)],
                        p0.at[pl.ds(sid * ROWS_T, ROWS_T)])

    @pl.when(cid == 1)
    def _():
        pltpu.sync_copy(shared.at[pl.ds(sid * ROWS_T, ROWS_T)],
                        p1.at[pl.ds(sid * ROWS_T, ROWS_T)])


_agg_kernel = functools.partial(
    pl.kernel,
    out_type=(jax.ShapeDtypeStruct((NROWS, D), jnp.bfloat16),
              jax.ShapeDtypeStruct((NROWS, D), jnp.bfloat16)),
    mesh=_mesh,
    scratch_types=[
        pltpu.VMEM((HK, CW), jnp.int32),
        pltpu.VMEM((HK, CW), jnp.int32),
        pltpu.VMEM((CW, D), jnp.bfloat16),
        pltpu.VMEM((CW, D), jnp.bfloat16),
        pltpu.VMEM((CW, D), jnp.bfloat16),
        pltpu.VMEM((CW, D), jnp.bfloat16),
        pltpu.VMEM_SHARED((NROWS, D), jnp.bfloat16),
        pltpu.SemaphoreType.DMA,
        pltpu.SemaphoreType.DMA,
        pltpu.SemaphoreType.DMA,
        pltpu.SemaphoreType.DMA,
        pltpu.SemaphoreType.DMA,
        pltpu.SemaphoreType.DMA,
        pltpu.SemaphoreType.DMA,
        pltpu.SemaphoreType.DMA,
    ],
    compiler_params=pltpu.CompilerParams(use_tc_tiling_on_sc=False),
)(_agg_body)


def _gather_body(h, bn, out, bidx, brows, sem):
    cid = lax.axis_index("c")
    sid = lax.axis_index("s")
    wid = cid * NS + sid
    base = wid * BPT
    pltpu.sync_copy(bn.at[pl.ds(base, BPT)], bidx)
    pltpu.async_copy(h.at[bidx], brows, sem).wait()
    pltpu.sync_copy(brows, out.at[pl.ds(base, BPT)])


_gather_kernel = functools.partial(
    pl.kernel,
    out_type=jax.ShapeDtypeStruct((B, D), jnp.float32),
    mesh=_mesh,
    scratch_types=[
        pltpu.VMEM((BPT,), jnp.int32),
        pltpu.VMEM((BPT, D), jnp.float32),
        pltpu.SemaphoreType.DMA,
    ],
)(_gather_body)


def _final_body(p0_ref, p1_ref, c1_ref, h_ref):
    s = p0_ref[:N].astype(jnp.float32) + p1_ref[:N].astype(jnp.float32)
    h_ref[...] = jnp.maximum(c1_ref[...][:, None] * s, 0.0)


def kernel(node_features, edge_index, batch_nodes, weight):
    src = edge_index[0]
    dst = edge_index[1]
    pad = EPAD - E
    ar = jnp.arange(pad, dtype=jnp.int32)
    src_p = jnp.concatenate([src, (ar * 13) % N]).reshape(NW, KCH, CW)
    dst_p = jnp.concatenate([dst, N + (ar % NTRASH)]).reshape(NW, KCH, CW)
    ones = jnp.ones((CW, DW), jnp.float32)
    z16 = jnp.zeros((NROWS, DW), jnp.float32)
    z = jnp.zeros((NROWS, D), jnp.bfloat16)

    degp = _deg_kernel(dst_p, z16, ones)

    y, c1 = pl.pallas_call(
        _scale_body,
        out_shape=(jax.ShapeDtypeStruct((N, D), jnp.bfloat16),
                   jax.ShapeDtypeStruct((N,), jnp.float32)),
    )(degp, node_features, weight)

    xb = node_features.astype(jnp.bfloat16)
    p0, p1 = _agg_kernel(y, xb, src_p, dst_p, z)

    h = pl.pallas_call(
        _final_body,
        out_shape=jax.ShapeDtypeStruct((N, D), jnp.float32),
    )(p0, p1, c1)

    return _gather_kernel(h, batch_nodes)
